# Initial kernel scaffold; baseline (speedup 1.0000x reference)
import functools
import math

import jax
import jax.numpy as jnp
from jax import lax
from jax.experimental import pallas as pl
from jax.experimental.pallas import tpu as pltpu
from jax.experimental.pallas import tpu_sc as plsc

N_CAT_ROWS = 1000001
EMB = 32
OUT_F = 32
TOTAL = 16384 * 50


_TR = 8192


def _transform_body(t_ref, w_ref, b_ref, o_ref):
    pid = pl.program_id(0)
    x = t_ref[...]
    rid = lax.broadcasted_iota(jnp.int32, x.shape, 0)
    x = jnp.where((pid * _TR + rid) == 0, 0.0, x)
    y = jnp.dot(x, w_ref[...], preferred_element_type=jnp.float32) + b_ref[...]
    o_ref[...] = jnp.where(y > 0, y, jnp.expm1(y))


def _transform_table(table, Wt, b2):
    grid = (math.ceil(N_CAT_ROWS / _TR),)
    return pl.pallas_call(
        _transform_body,
        grid=grid,
        in_specs=[
            pl.BlockSpec((_TR, EMB), lambda i: (i, 0)),
            pl.BlockSpec((EMB, OUT_F), lambda i: (0, 0)),
            pl.BlockSpec((1, OUT_F), lambda i: (0, 0)),
        ],
        out_specs=pl.BlockSpec((_TR, OUT_F), lambda i: (i, 0)),
        out_shape=jax.ShapeDtypeStruct((N_CAT_ROWS, OUT_F), jnp.float32),
    )(table, Wt, b2)


_INFO = plsc.get_sparse_core_info()
_NC = _INFO.num_cores
_NS = _INFO.num_subcores
_NW = _NC * _NS
_PER_W = TOTAL // _NW
_IDX_ROWS = 10
_CHUNK = _IDX_ROWS * 128
_N_CHUNKS = _PER_W // _CHUNK


def _gather_kernel(t_hbm, idx_hbm, out_hbm, idx_v, rows_v, sem):
    wid = lax.axis_index("s") * _NC + lax.axis_index("c")
    idx_row0 = wid * (_PER_W // 128)
    out0 = wid * _PER_W

    def body(g, carry):
        pltpu.sync_copy(idx_hbm.at[pl.ds(idx_row0 + g * _IDX_ROWS, _IDX_ROWS)],
                        idx_v)
        handles = [
            pltpu.async_copy(t_hbm.at[idx_v.at[j]],
                             rows_v.at[pl.ds(j * 128, 128)], sem)
            for j in range(_IDX_ROWS)
        ]
        for h in handles:
            h.wait()
        pltpu.sync_copy(rows_v, out_hbm.at[pl.ds(out0 + g * _CHUNK, _CHUNK)])
        return carry

    lax.fori_loop(0, _N_CHUNKS, body, 0)


def _sc_gather(t, idx2d):
    mesh = plsc.VectorSubcoreMesh(core_axis_name="c", subcore_axis_name="s")
    f = functools.partial(
        pl.kernel,
        mesh=mesh,
        out_type=jax.ShapeDtypeStruct((TOTAL, OUT_F), jnp.float32),
        scratch_types=[
            pltpu.VMEM((_IDX_ROWS, 128), jnp.int32),
            pltpu.VMEM((_CHUNK, OUT_F), jnp.float32),
            pltpu.SemaphoreType.DMA,
        ],
    )(_gather_kernel)
    return f(t, idx2d)


def kernel(sequences, emb_table, W, b):
    B, L = sequences.shape
    t = _transform_table(emb_table, W.T, b.reshape(1, OUT_F))
    idx2d = sequences.reshape(TOTAL // 128, 128).astype(jnp.int32)
    out = _sc_gather(t, idx2d)
    return out.reshape(B, L, OUT_F)


# speedup vs baseline: 9.7373x; 9.7373x over previous
import functools
import math

import jax
import jax.numpy as jnp
from jax import lax
from jax.experimental import pallas as pl
from jax.experimental.pallas import tpu as pltpu
from jax.experimental.pallas import tpu_sc as plsc

N_CAT_ROWS = 1000001
EMB = 32
OUT_F = 32
TOTAL = 16384 * 50


_TR = 8192


def _transform_body(t_ref, w_ref, b_ref, o_ref):
    pid = pl.program_id(0)
    x = t_ref[...]
    rid = lax.broadcasted_iota(jnp.int32, x.shape, 0)
    x = jnp.where((pid * _TR + rid) == 0, 0.0, x)
    y = jnp.dot(x, w_ref[...], preferred_element_type=jnp.float32) + b_ref[...]
    o_ref[...] = jnp.where(y > 0, y, jnp.exp(y) - 1.0)


def _transform_table(table, Wt, b2):
    grid = (math.ceil(N_CAT_ROWS / _TR),)
    return pl.pallas_call(
        _transform_body,
        grid=grid,
        in_specs=[
            pl.BlockSpec((_TR, EMB), lambda i: (i, 0)),
            pl.BlockSpec((EMB, OUT_F), lambda i: (0, 0)),
            pl.BlockSpec((1, OUT_F), lambda i: (0, 0)),
        ],
        out_specs=pl.BlockSpec((_TR, OUT_F), lambda i: (i, 0)),
        out_shape=jax.ShapeDtypeStruct((N_CAT_ROWS, OUT_F), jnp.float32),
    )(table, Wt, b2)


_INFO = plsc.get_sparse_core_info()
_NC = _INFO.num_cores
_NS = _INFO.num_subcores
_NW = _NC * _NS
_PER_W = TOTAL // _NW
_IDX_ROWS = 8
_CHUNK = _IDX_ROWS * 128
_N_CHUNKS = _PER_W // _CHUNK


def _gather_kernel(t_hbm, idx_hbm, out_hbm, idx_v, rows_v, sem):
    wid = lax.axis_index("s") * _NC + lax.axis_index("c")
    idx_row0 = wid * (_PER_W // 128)
    out0 = wid * _PER_W

    def body(g, carry):
        pltpu.sync_copy(idx_hbm.at[pl.ds(idx_row0 + g * _IDX_ROWS, _IDX_ROWS)],
                        idx_v)
        handles = [
            pltpu.async_copy(t_hbm.at[idx_v.at[j]],
                             rows_v.at[pl.ds(j * 128, 128)], sem)
            for j in range(_IDX_ROWS)
        ]
        for h in handles:
            h.wait()
        pltpu.sync_copy(rows_v, out_hbm.at[pl.ds(out0 + g * _CHUNK, _CHUNK)])
        return carry

    lax.fori_loop(0, _N_CHUNKS, body, 0)


def _sc_gather(t, idx2d):
    mesh = plsc.VectorSubcoreMesh(core_axis_name="c", subcore_axis_name="s")
    f = functools.partial(
        pl.kernel,
        mesh=mesh,
        out_type=jax.ShapeDtypeStruct((TOTAL, OUT_F), jnp.float32),
        scratch_types=[
            pltpu.VMEM((_IDX_ROWS, 128), jnp.int32),
            pltpu.VMEM((_CHUNK, OUT_F), jnp.float32),
            pltpu.SemaphoreType.DMA,
        ],
        compiler_params=pltpu.CompilerParams(use_tc_tiling_on_sc=False),
    )(_gather_kernel)
    return f(t, idx2d)


def kernel(sequences, emb_table, W, b):
    B, L = sequences.shape
    t = _transform_table(emb_table, W.T, b.reshape(1, OUT_F))
    idx2d = sequences.reshape(TOTAL // 128, 128).astype(jnp.int32)
    out = _sc_gather(t, idx2d)
    return out.reshape(B, L, OUT_F)


# speedup vs baseline: 10.8213x; 1.1113x over previous
import functools
import math

import jax
import jax.numpy as jnp
from jax import lax
from jax.experimental import pallas as pl
from jax.experimental.pallas import tpu as pltpu
from jax.experimental.pallas import tpu_sc as plsc

N_CAT_ROWS = 1000001
EMB = 32
OUT_F = 32
TOTAL = 16384 * 50


_TC = 1024
_GRID = math.ceil(N_CAT_ROWS / _TC)
_NPAD = _GRID * _TC


def _transform_body(xt_ref, w_ref, b_ref, o_ref):
    pid = pl.program_id(0)
    x = xt_ref[...]
    col = lax.broadcasted_iota(jnp.int32, x.shape, 1) + pid * _TC
    x = jnp.where(col == 0, 0.0, x)
    y = lax.dot_general(x, w_ref[...], (((0,), (0,)), ((), ())),
                        preferred_element_type=jnp.float32) + b_ref[...]
    y = jnp.where(y > 0, y, jnp.exp(y) - 1.0)
    o_ref[...] = jnp.concatenate(
        [y, jnp.zeros((_TC, 128 - OUT_F), jnp.float32)], axis=1)


def _transform_table(xt, Wt, b2):
    return pl.pallas_call(
        _transform_body,
        grid=(_GRID,),
        in_specs=[
            pl.BlockSpec((EMB, _TC), lambda i: (0, i)),
            pl.BlockSpec((EMB, OUT_F), lambda i: (0, 0)),
            pl.BlockSpec((1, OUT_F), lambda i: (0, 0)),
        ],
        out_specs=pl.BlockSpec((_TC, 128), lambda i: (i, 0)),
        out_shape=jax.ShapeDtypeStruct((_NPAD, 128), jnp.float32),
    )(xt, Wt, b2)


_INFO = plsc.get_sparse_core_info()
_NC = _INFO.num_cores
_NS = _INFO.num_subcores
_NW = _NC * _NS
_PER_W = TOTAL // _NW
_IDX_ROWS = 8
_CHUNK = _IDX_ROWS * 128
_N_CHUNKS = _PER_W // _CHUNK


def _gather_kernel(t_hbm, idx_hbm, out_hbm, idx_v, rows_v, sem):
    wid = lax.axis_index("s") * _NC + lax.axis_index("c")
    idx_row0 = wid * (_PER_W // 128)
    out0 = wid * _PER_W

    def body(g, carry):
        pltpu.sync_copy(idx_hbm.at[pl.ds(idx_row0 + g * _IDX_ROWS, _IDX_ROWS)],
                        idx_v)
        handles = [
            pltpu.async_copy(t_hbm.at[idx_v.at[j]],
                             rows_v.at[pl.ds(j * 128, 128)], sem)
            for j in range(_IDX_ROWS)
        ]
        for h in handles:
            h.wait()
        pltpu.sync_copy(rows_v, out_hbm.at[pl.ds(out0 + g * _CHUNK, _CHUNK)])
        return carry

    lax.fori_loop(0, _N_CHUNKS, body, 0)


def _sc_gather(t4, idx2d):
    mesh = plsc.VectorSubcoreMesh(core_axis_name="c", subcore_axis_name="s")
    f = functools.partial(
        pl.kernel,
        mesh=mesh,
        out_type=jax.ShapeDtypeStruct((TOTAL, OUT_F), jnp.float32),
        scratch_types=[
            pltpu.VMEM((_IDX_ROWS, 128), jnp.int32),
            pltpu.VMEM((_CHUNK, OUT_F), jnp.float32),
            pltpu.SemaphoreType.DMA,
        ],
        compiler_params=pltpu.CompilerParams(use_tc_tiling_on_sc=False),
    )(_gather_kernel)
    return f(t4, idx2d)


def kernel(sequences, emb_table, W, b):
    B, L = sequences.shape
    t128 = _transform_table(emb_table.T, W.T, b.reshape(1, OUT_F))
    t4 = t128.reshape(_NPAD * 4, OUT_F)
    idx2d = (sequences.astype(jnp.int32) * 4).reshape(TOTAL // 128, 128)
    out = _sc_gather(t4, idx2d)
    return out.reshape(B, L, OUT_F)


# speedup vs baseline: 28.3373x; 2.6187x over previous
import functools
import math

import jax
import jax.numpy as jnp
from jax import lax
from jax.experimental import pallas as pl
from jax.experimental.pallas import tpu as pltpu
from jax.experimental.pallas import tpu_sc as plsc

N_CAT_ROWS = 1000001
EMB = 32
OUT_F = 32
TOTAL = 16384 * 50


_TC = 8192
_GRID = math.ceil(N_CAT_ROWS / _TC)
_NPAD = _GRID * _TC


def _transform_body(xt_ref, w_ref, b_ref, o_ref):
    pid = pl.program_id(0)
    x = xt_ref[...]
    col = lax.broadcasted_iota(jnp.int32, x.shape, 1) + pid * _TC
    x = jnp.where(col == 0, 0.0, x)
    y = lax.dot_general(x, w_ref[...], (((0,), (0,)), ((), ())),
                        preferred_element_type=jnp.float32) + b_ref[...]
    y = jnp.where(y > 0, y, jnp.exp(y) - 1.0)
    o_ref[...] = jnp.concatenate(
        [y, jnp.zeros((_TC, 128 - OUT_F), jnp.float32)], axis=1)


def _transform_table(xt, Wt, b2):
    return pl.pallas_call(
        _transform_body,
        grid=(_GRID,),
        in_specs=[
            pl.BlockSpec((EMB, _TC), lambda i: (0, i)),
            pl.BlockSpec((EMB, OUT_F), lambda i: (0, 0)),
            pl.BlockSpec((1, OUT_F), lambda i: (0, 0)),
        ],
        out_specs=pl.BlockSpec((_TC, 128), lambda i: (i, 0)),
        out_shape=jax.ShapeDtypeStruct((_NPAD, 128), jnp.float32),
    )(xt, Wt, b2)


_INFO = plsc.get_sparse_core_info()
_NC = _INFO.num_cores
_NS = _INFO.num_subcores
_NW = _NC * _NS
_PER_W = TOTAL // _NW
_IDX_ROWS = 8
_CHUNK = _IDX_ROWS * 128
_N_CHUNKS = _PER_W // _CHUNK


def _gather_kernel(t_hbm, idx_hbm, out_hbm, idx_v, rows_v, sem):
    wid = lax.axis_index("s") * _NC + lax.axis_index("c")
    idx_row0 = wid * (_PER_W // 128)
    out0 = wid * _PER_W

    def body(g, carry):
        pltpu.sync_copy(idx_hbm.at[pl.ds(idx_row0 + g * _IDX_ROWS, _IDX_ROWS)],
                        idx_v)
        handles = [
            pltpu.async_copy(t_hbm.at[idx_v.at[j]],
                             rows_v.at[pl.ds(j * 128, 128)], sem)
            for j in range(_IDX_ROWS)
        ]
        for h in handles:
            h.wait()
        pltpu.sync_copy(rows_v, out_hbm.at[pl.ds(out0 + g * _CHUNK, _CHUNK)])
        return carry

    lax.fori_loop(0, _N_CHUNKS, body, 0)


def _sc_gather(t4, idx2d):
    mesh = plsc.VectorSubcoreMesh(core_axis_name="c", subcore_axis_name="s")
    f = functools.partial(
        pl.kernel,
        mesh=mesh,
        out_type=jax.ShapeDtypeStruct((TOTAL, OUT_F), jnp.float32),
        scratch_types=[
            pltpu.VMEM((_IDX_ROWS, 128), jnp.int32),
            pltpu.VMEM((_CHUNK, OUT_F), jnp.float32),
            pltpu.SemaphoreType.DMA,
        ],
        compiler_params=pltpu.CompilerParams(use_tc_tiling_on_sc=False),
    )(_gather_kernel)
    return f(t4, idx2d)


def kernel(sequences, emb_table, W, b):
    B, L = sequences.shape
    t128 = _transform_table(emb_table.T, W.T, b.reshape(1, OUT_F))
    t4 = t128.reshape(_NPAD * 4, OUT_F)
    idx2d = (sequences.T.astype(jnp.int32) * 4).reshape(TOTAL // 128, 128)
    out = _sc_gather(t4, idx2d)
    return out.reshape(L, B, OUT_F).transpose(1, 0, 2)
